# baseline (speedup 1.0000x reference)
import jax
import jax.numpy as jnp
from jax import lax
from jax.experimental import pallas as pl
from jax.experimental.pallas import tpu as pltpu
from jax.experimental.pallas import tpu_sc as plsc

NC = 2
NS = 16
NW = NC * NS
L = 16

BATCH = 4096
SEQ = 200
D = 64
TOTAL = BATCH * SEQ
PER_W = TOTAL // NW
CHUNK = 128
NCHUNK = PER_W // CHUNK


def _embd_body(wte_hbm, idx_hbm, out_hbm, idx_v, rows_v, sem):
    wid = lax.axis_index("s") * NC + lax.axis_index("c")
    base = wid * PER_W
    pltpu.sync_copy(idx_hbm.at[wid], idx_v)

    @pl.loop(0, NCHUNK)
    def _chunk(j):
        pltpu.async_copy(wte_hbm.at[idx_v.at[j]], rows_v, sem).wait()

        @pl.loop(0, CHUNK)
        def _row(r):
            for c4 in range(D // L):
                sl = pl.ds(c4 * L, L)
                rows_v[r, sl] = rows_v[r, sl] + 1.0

        pltpu.sync_copy(rows_v, out_hbm.at[pl.ds(base + j * CHUNK, CHUNK)])


@jax.jit
def kernel(idx, wte):
    idx3 = idx.astype(jnp.int32).reshape(NW, NCHUNK, CHUNK)
    mesh = plsc.VectorSubcoreMesh(
        core_axis_name="c", subcore_axis_name="s",
        num_cores=NC, num_subcores=NS)
    out = pl.kernel(
        _embd_body,
        out_type=jax.ShapeDtypeStruct((TOTAL, D), jnp.float32),
        mesh=mesh,
        compiler_params=pltpu.CompilerParams(use_tc_tiling_on_sc=False),
        scratch_types=[
            pltpu.VMEM((NCHUNK, CHUNK), jnp.int32),
            pltpu.VMEM((CHUNK, D), jnp.float32),
            pltpu.SemaphoreType.DMA,
        ],
    )(wte, idx3)
    return out.reshape(BATCH, SEQ, D)


# speedup vs baseline: 1.2120x; 1.2120x over previous
import jax
import jax.numpy as jnp
from jax import lax
from jax.experimental import pallas as pl
from jax.experimental.pallas import tpu as pltpu
from jax.experimental.pallas import tpu_sc as plsc

NC = 2
NS = 16
NW = NC * NS
L = 16

BATCH = 4096
SEQ = 200
D = 64
TOTAL = BATCH * SEQ
PER_W = TOTAL // NW
CHUNK = 128
NCHUNK = PER_W // CHUNK
NBUF = 4


def _embd_body(wte_hbm, idx_hbm, out_hbm, idx_v,
               g0, g1, g2, g3, s0, s1, s2, s3,
               gs0, gs1, gs2, gs3, ss0, ss1, ss2, ss3):
    gbuf = [g0, g1, g2, g3]
    sbuf = [s0, s1, s2, s3]
    gsem = [gs0, gs1, gs2, gs3]
    ssem = [ss0, ss1, ss2, ss3]

    wid = lax.axis_index("s") * NC + lax.axis_index("c")
    base = wid * PER_W
    pltpu.sync_copy(idx_hbm.at[wid], idx_v)

    def fire_gather(j, b):
        pltpu.async_copy(wte_hbm.at[idx_v.at[j]], gbuf[b], gsem[b])

    def wait_gather(b):
        pltpu.make_async_copy(wte_hbm.at[pl.ds(0, CHUNK)], gbuf[b], gsem[b]).wait()

    def fire_store(j, b):
        pltpu.async_copy(sbuf[b], out_hbm.at[pl.ds(base + j * CHUNK, CHUNK)], ssem[b])

    def wait_store(b):
        pltpu.make_async_copy(sbuf[b], out_hbm.at[pl.ds(base, CHUNK)], ssem[b]).wait()

    def add_one(b):
        gb, sb = gbuf[b], sbuf[b]

        @plsc.parallel_loop(0, CHUNK, unroll=4)
        def _row(r):
            for c4 in range(D // L):
                sl = pl.ds(c4 * L, L)
                sb[r, sl] = gb[r, sl] + 1.0

    for b in range(NBUF):
        fire_gather(b, b)

    for b in range(NBUF):
        wait_gather(b)
        add_one(b)
        fire_gather(b + NBUF, b)
        fire_store(b, b)

    @pl.loop(NBUF, NCHUNK - NBUF, step=NBUF)
    def _outer(j0):
        for b in range(NBUF):
            j = j0 + b
            wait_gather(b)
            wait_store(b)
            add_one(b)
            fire_gather(j + NBUF, b)
            fire_store(j, b)

    for b in range(NBUF):
        j = NCHUNK - NBUF + b
        wait_gather(b)
        wait_store(b)
        add_one(b)
        fire_store(j, b)

    for b in range(NBUF):
        wait_store(b)


@jax.jit
def kernel(idx, wte):
    idx3 = idx.astype(jnp.int32).reshape(NW, NCHUNK, CHUNK)
    mesh = plsc.VectorSubcoreMesh(
        core_axis_name="c", subcore_axis_name="s",
        num_cores=NC, num_subcores=NS)
    out = pl.kernel(
        _embd_body,
        out_type=jax.ShapeDtypeStruct((TOTAL, D), jnp.float32),
        mesh=mesh,
        compiler_params=pltpu.CompilerParams(use_tc_tiling_on_sc=False),
        scratch_types=(
            [pltpu.VMEM((NCHUNK, CHUNK), jnp.int32)]
            + [pltpu.VMEM((CHUNK, D), jnp.float32) for _ in range(2 * NBUF)]
            + [pltpu.SemaphoreType.DMA for _ in range(2 * NBUF)]
        ),
    )(wte, idx3)
    return out.reshape(BATCH, SEQ, D)


# speedup vs baseline: 1.9824x; 1.6357x over previous
import jax
import jax.numpy as jnp
from jax import lax
from jax.experimental import pallas as pl
from jax.experimental.pallas import tpu as pltpu
from jax.experimental.pallas import tpu_sc as plsc

NC = 2
NS = 16
NW = NC * NS
L = 16

BATCH = 4096
SEQ = 200
D = 64
CHUNK = 128
NBB = BATCH // CHUNK
NPAIR = SEQ * NBB
PER_W = NPAIR // NW
NBUF = 4
C8 = D // 8


def _embd_body(wte_hbm, idx_hbm, out_hbm, idx_v,
               g0, g1, g2, g3, s0, s1, s2, s3,
               gs0, gs1, gs2, gs3, ss0, ss1, ss2, ss3):
    gbuf = [g0, g1, g2, g3]
    sbuf = [s0, s1, s2, s3]
    gsem = [gs0, gs1, gs2, gs3]
    ssem = [ss0, ss1, ss2, ss3]

    wid = lax.axis_index("s") * NC + lax.axis_index("c")
    base = wid * PER_W
    pltpu.sync_copy(idx_hbm.at[wid], idx_v)

    iota = lax.iota(jnp.int32, L)
    c8vecs = [(iota + 16 * g) // 8 for g in range(D // L)]
    c1vecs = [(iota + 16 * g) % 8 for g in range(D // L)]

    def fire_gather(j, b):
        pltpu.async_copy(wte_hbm.at[idx_v.at[j]], gbuf[b], gsem[b])

    def wait_gather(b):
        pltpu.make_async_copy(wte_hbm.at[pl.ds(0, CHUNK)], gbuf[b], gsem[b]).wait()

    def fire_store(j, b):
        p = base + j
        s = p // NBB
        bb = p % NBB
        pltpu.async_copy(
            sbuf[b].at[:, :, pl.ds(0, CHUNK)], out_hbm.at[s, :, bb], ssem[b])

    def wait_store(b):
        pltpu.make_async_copy(
            sbuf[b].at[:, :, pl.ds(0, CHUNK)], out_hbm.at[0, :, 0], ssem[b]).wait()

    def transform(b):
        gb, sb = gbuf[b], sbuf[b]

        @plsc.parallel_loop(0, CHUNK, unroll=2)
        def _row(b1):
            bvec = jnp.zeros((L,), jnp.int32) + b1
            for g in range(D // L):
                v = gb[b1, pl.ds(16 * g, L)] + 1.0
                plsc.store_scatter(sb, [c8vecs[g], c1vecs[g], bvec], v)

    for b in range(NBUF):
        fire_gather(b, b)

    for b in range(NBUF):
        wait_gather(b)
        transform(b)
        fire_gather(b + NBUF, b)
        fire_store(b, b)

    @pl.loop(NBUF, PER_W - NBUF, step=NBUF)
    def _outer(j0):
        for b in range(NBUF):
            j = j0 + b
            wait_gather(b)
            wait_store(b)
            transform(b)
            fire_gather(j + NBUF, b)
            fire_store(j, b)

    for b in range(NBUF):
        j = PER_W - NBUF + b
        wait_gather(b)
        wait_store(b)
        transform(b)
        fire_store(j, b)

    for b in range(NBUF):
        wait_store(b)


@jax.jit
def kernel(idx, wte):
    idxf = jnp.transpose(idx).reshape(NPAIR, CHUNK).astype(jnp.int32)
    idx3 = idxf.reshape(NW, PER_W, CHUNK)
    mesh = plsc.VectorSubcoreMesh(
        core_axis_name="c", subcore_axis_name="s",
        num_cores=NC, num_subcores=NS)
    out5 = pl.kernel(
        _embd_body,
        out_type=jax.ShapeDtypeStruct((SEQ, C8, NBB, 8, CHUNK), jnp.float32),
        mesh=mesh,
        compiler_params=pltpu.CompilerParams(
            use_tc_tiling_on_sc=False, needs_layout_passes=False),
        scratch_types=(
            [pltpu.VMEM((PER_W, CHUNK), jnp.int32)]
            + [pltpu.VMEM((CHUNK, D), jnp.float32) for _ in range(NBUF)]
            + [pltpu.VMEM((C8, 8, CHUNK + 1), jnp.float32) for _ in range(NBUF)]
            + [pltpu.SemaphoreType.DMA for _ in range(2 * NBUF)]
        ),
    )(wte, idx3)
    return jnp.transpose(out5, (2, 4, 0, 1, 3)).reshape(BATCH, SEQ, D)
